# baseline (speedup 1.0000x reference)
import functools

import jax
import jax.numpy as jnp
from jax import lax
from jax.experimental import pallas as pl
from jax.experimental.pallas import tpu as pltpu
from jax.experimental.pallas import tpu_sc as plsc

_NC = 2
_NS = 16
_L = 16
_NW = _NC * _NS
_B = 16384
_D = 64
_BPW = _B // _NW
_CHUNK = 128
_NCHUNK = _BPW // _CHUNK


def _sqrt16(x):
    xs = jnp.maximum(x, jnp.float32(1e-30))
    i = plsc.bitcast(xs, jnp.int32)
    i = jnp.int32(0x5F3759DF) - (i >> 1)
    y = plsc.bitcast(i, jnp.float32)
    half = xs * jnp.float32(0.5)
    for _ in range(3):
        y = y * (jnp.float32(1.5) - half * y * y)
    return xs * y


def _transe_body(head_hbm, rel_hbm, tail_hbm, ent_hbm, relemb_hbm, out_hbm,
                 hidx, ridx, tidx, hpidx, rpidx, tpidx,
                 hrows, rrows, trows, tscr, oloc, sem):
    wid = lax.axis_index("s") * _NC + lax.axis_index("c")
    base = wid * _BPW

    pltpu.sync_copy(head_hbm.at[pl.ds(base, _BPW)], hidx)
    pltpu.sync_copy(rel_hbm.at[pl.ds(base, _BPW)], ridx)
    pltpu.sync_copy(tail_hbm.at[pl.ds(base, _BPW)], tidx)

    for j in range(_BPW // _L):
        ds = pl.ds(j * _L, _L)
        hpidx[ds] = hidx[ds] >> 1
        rpidx[ds] = ridx[ds] >> 1
        tpidx[ds] = tidx[ds] >> 1

    lanes = lax.iota(jnp.int32, _L)

    def do_chunk(c, slot):
        sl = pl.ds(c * _CHUNK, _CHUNK)
        cps = [
            pltpu.async_copy(ent_hbm.at[hpidx.at[sl]], hrows.at[slot], sem),
            pltpu.async_copy(relemb_hbm.at[rpidx.at[sl]], rrows.at[slot], sem),
            pltpu.async_copy(ent_hbm.at[tpidx.at[sl]], trows.at[slot], sem),
        ]
        return cps

    def compute_chunk(c, slot):
        cbase = c * _CHUNK

        def block_body(b, carry):
            rbase = b * _L
            gds = pl.ds(cbase + rbase, _L)
            hvec = hidx[gds] & 1
            rvec = ridx[gds] & 1
            tvec = tidx[gds] & 1
            for r in range(_L):
                row = rbase + r
                hoff = hvec[r] * _D
                roff = rvec[r] * _D
                toff = tvec[r] * _D
                acc = None
                for cc in range(_D // _L):
                    hv = hrows[slot, row, pl.ds(hoff + cc * _L, _L)]
                    rv = rrows[slot, row, pl.ds(roff + cc * _L, _L)]
                    tv = trows[slot, row, pl.ds(toff + cc * _L, _L)]
                    d = hv + rv - tv
                    sq = d * d
                    acc = sq if acc is None else acc + sq
                tscr[pl.ds(r * _L, _L)] = acc
            s = None
            for col in range(_L):
                g = plsc.load_gather(tscr, [lanes * _L + col])
                s = g if s is None else s + g
            oloc[pl.ds(cbase + rbase, _L)] = _sqrt16(s)
            return carry

        lax.fori_loop(0, _CHUNK // _L, block_body, 0)

    inflight = do_chunk(0, 0)
    for c in range(_NCHUNK):
        nxt = do_chunk(c + 1, (c + 1) % 2) if c + 1 < _NCHUNK else []
        for cp in inflight:
            cp.wait()
        compute_chunk(c, c % 2)
        inflight = nxt

    pltpu.sync_copy(oloc, out_hbm.at[pl.ds(base, _BPW)])


_transe = functools.partial(
    pl.kernel,
    out_type=jax.ShapeDtypeStruct((_B,), jnp.float32),
    mesh=plsc.VectorSubcoreMesh(core_axis_name="c", subcore_axis_name="s",
                                num_cores=_NC, num_subcores=_NS),
    compiler_params=pltpu.CompilerParams(needs_layout_passes=False,
                                         use_tc_tiling_on_sc=True),
    scratch_types=[
        pltpu.VMEM((_BPW,), jnp.int32),
        pltpu.VMEM((_BPW,), jnp.int32),
        pltpu.VMEM((_BPW,), jnp.int32),
        pltpu.VMEM((_BPW,), jnp.int32),
        pltpu.VMEM((_BPW,), jnp.int32),
        pltpu.VMEM((_BPW,), jnp.int32),
        pltpu.VMEM((2, _CHUNK, 2 * _D), jnp.float32),
        pltpu.VMEM((2, _CHUNK, 2 * _D), jnp.float32),
        pltpu.VMEM((2, _CHUNK, 2 * _D), jnp.float32),
        pltpu.VMEM((_L * _L,), jnp.float32),
        pltpu.VMEM((_BPW,), jnp.float32),
        pltpu.SemaphoreType.DMA,
    ],
)(_transe_body)


def kernel(head, rel, tail, ent_emb, rel_emb):
    h = head.astype(jnp.int32)
    r = rel.astype(jnp.int32)
    t = tail.astype(jnp.int32)
    ent2 = ent_emb.reshape(ent_emb.shape[0] // 2, 2 * _D)
    rel2 = rel_emb.reshape(rel_emb.shape[0] // 2, 2 * _D)
    return _transe(h, r, t, ent2, rel2)


# speedup vs baseline: 1.1003x; 1.1003x over previous
import functools

import jax
import jax.numpy as jnp
from jax import lax
from jax.experimental import pallas as pl
from jax.experimental.pallas import tpu as pltpu
from jax.experimental.pallas import tpu_sc as plsc

_NC = 2
_NS = 16
_L = 16
_NW = _NC * _NS
_B = 16384
_D = 64
_BPW = _B // _NW
_CHUNK = 128
_NCHUNK = _BPW // _CHUNK


def _sqrt16(x):
    xs = jnp.maximum(x, jnp.float32(1e-30))
    i = plsc.bitcast(xs, jnp.int32)
    i = jnp.int32(0x5F3759DF) - (i >> 1)
    y = plsc.bitcast(i, jnp.float32)
    half = xs * jnp.float32(0.5)
    for _ in range(3):
        y = y * (jnp.float32(1.5) - half * y * y)
    return xs * y


def _transe_body(head_hbm, rel_hbm, tail_hbm, ent_hbm, relemb_hbm, out_hbm,
                 hidx, ridx, tidx, hpidx, rpidx, tpidx,
                 hrows, rrows, trows, tscr, oloc, sem):
    wid = lax.axis_index("s") * _NC + lax.axis_index("c")
    base = wid * _BPW

    pltpu.sync_copy(head_hbm.at[pl.ds(base, _BPW)], hidx)
    pltpu.sync_copy(rel_hbm.at[pl.ds(base, _BPW)], ridx)
    pltpu.sync_copy(tail_hbm.at[pl.ds(base, _BPW)], tidx)

    for j in range(_BPW // _L):
        ds = pl.ds(j * _L, _L)
        hpidx[ds] = hidx[ds] >> 1
        rpidx[ds] = ridx[ds] >> 1
        tpidx[ds] = tidx[ds] >> 1

    lanes = lax.iota(jnp.int32, _L)

    def do_chunk(c, slot):
        sl = pl.ds(c * _CHUNK, _CHUNK)
        cps = [
            pltpu.async_copy(ent_hbm.at[hpidx.at[sl]], hrows.at[slot], sem),
            pltpu.async_copy(relemb_hbm.at[rpidx.at[sl]], rrows.at[slot], sem),
            pltpu.async_copy(ent_hbm.at[tpidx.at[sl]], trows.at[slot], sem),
        ]
        return cps

    def compute_chunk(c, slot):
        cbase = c * _CHUNK

        def block_body(b, carry):
            rbase = b * _L
            gds = pl.ds(cbase + rbase, _L)
            hvec = hidx[gds] & 1
            rvec = ridx[gds] & 1
            tvec = tidx[gds] & 1
            for r in range(_L):
                row = rbase + r
                hoff = hvec[r] * _D
                roff = rvec[r] * _D
                toff = tvec[r] * _D
                acc = None
                for cc in range(_D // _L):
                    hv = hrows[slot, row, pl.ds(hoff + cc * _L, _L)]
                    rv = rrows[slot, row, pl.ds(roff + cc * _L, _L)]
                    tv = trows[slot, row, pl.ds(toff + cc * _L, _L)]
                    d = hv + rv - tv
                    sq = d * d
                    acc = sq if acc is None else acc + sq
                tscr[pl.ds(r * _L, _L)] = acc
            s = None
            for col in range(_L):
                g = plsc.load_gather(tscr, [lanes * _L + col])
                s = g if s is None else s + g
            oloc[pl.ds(cbase + rbase, _L)] = _sqrt16(s)
            return carry

        lax.fori_loop(0, _CHUNK // _L, block_body, 0)

    inflight = do_chunk(0, 0)
    for c in range(_NCHUNK):
        nxt = do_chunk(c + 1, (c + 1) % 2) if c + 1 < _NCHUNK else []
        for cp in inflight:
            cp.wait()
        compute_chunk(c, c % 2)
        inflight = nxt

    pltpu.sync_copy(oloc, out_hbm.at[pl.ds(base, _BPW)])


_transe = functools.partial(
    pl.kernel,
    out_type=jax.ShapeDtypeStruct((_B,), jnp.float32),
    mesh=plsc.VectorSubcoreMesh(core_axis_name="c", subcore_axis_name="s",
                                num_cores=_NC, num_subcores=_NS),
    compiler_params=pltpu.CompilerParams(needs_layout_passes=False,
                                         use_tc_tiling_on_sc=True),
    scratch_types=[
        pltpu.VMEM((_BPW,), jnp.int32),
        pltpu.VMEM((_BPW,), jnp.int32),
        pltpu.VMEM((_BPW,), jnp.int32),
        pltpu.VMEM((_BPW,), jnp.int32),
        pltpu.VMEM((_BPW,), jnp.int32),
        pltpu.VMEM((_BPW,), jnp.int32),
        pltpu.VMEM((2, _CHUNK, 2 * _D), jnp.float32),
        pltpu.VMEM((2, _CHUNK, 2 * _D), jnp.float32),
        pltpu.VMEM((2, _CHUNK, 2 * _D), jnp.float32),
        pltpu.VMEM((_L * _L,), jnp.float32),
        pltpu.VMEM((_BPW,), jnp.float32),
        pltpu.SemaphoreType.DMA,
    ],
)(_transe_body)


_XL = 2048


def _xpose_body(p_ref, o_ref):
    x = p_ref[...]
    xt = x.T.reshape(_XL // 2, 2, _D)
    o_ref[:, 0:_D] = xt[:, 0, :]
    o_ref[:, _D:2 * _D] = xt[:, 1, :]


def _xpose(p, n_ent):
    grid = (n_ent + _XL - 1) // _XL
    return pl.pallas_call(
        _xpose_body,
        grid=(grid,),
        in_specs=[pl.BlockSpec((_D, _XL), lambda i: (0, i))],
        out_specs=pl.BlockSpec((_XL // 2, 2 * _D), lambda i: (i, 0)),
        out_shape=jax.ShapeDtypeStruct((n_ent // 2, 2 * _D), jnp.float32),
    )(p)


def kernel(head, rel, tail, ent_emb, rel_emb):
    h = head.astype(jnp.int32)
    r = rel.astype(jnp.int32)
    t = tail.astype(jnp.int32)
    ent2 = _xpose(ent_emb.T, ent_emb.shape[0])
    rel2 = rel_emb.reshape(rel_emb.shape[0] // 2, 2 * _D)
    return _transe(h, r, t, ent2, rel2)
